# Initial kernel scaffold; baseline (speedup 1.0000x reference)
import jax
import jax.numpy as jnp
from jax.experimental import pallas as pl


def kernel(node_types, edge_index, edge_dist, embed, W_rbf, W1, Wg, We, Wout):
    raise NotImplementedError("write your pallas kernel here")


# speedup vs baseline: 1.4161x; 1.4161x over previous
import functools

import jax
import jax.numpy as jnp
from jax import lax
from jax.experimental import pallas as pl
from jax.experimental.pallas import tpu as pltpu
from jax.experimental.pallas import tpu_sc as plsc

N = 10000
E = 160000
D = 64
MAX_N = 9
NBLOCKS = 4
N_ELEM = 10
CUTOFF = 5.0

NC = 2
NS = 16
NW = NC * NS

CHUNK = 128
E_PAD = 163840
EPW = E_PAD // NW
NCH = EPW // CHUNK
N_PAD = 10240
RPT = N_PAD // NS

EBLK = 4096

def _sc_mesh():
    return plsc.VectorSubcoreMesh(core_axis_name="c", subcore_axis_name="s",
                                  num_cores=NC, num_subcores=NS)


def _gather_body(h_hbm, src_hbm, dst_hbm, hs_hbm, hd_hbm,
                 idx_v, rows_v, sem):
    c = lax.axis_index("c")
    s = lax.axis_index("s")
    w = c * NS + s
    base = w * EPW

    def chunk(j, _):
        off = base + j * CHUNK
        pltpu.sync_copy(src_hbm.at[w, j], idx_v)
        pltpu.async_copy(h_hbm.at[idx_v], rows_v, sem).wait()
        pltpu.sync_copy(rows_v, hs_hbm.at[pl.ds(off, CHUNK)])
        pltpu.sync_copy(dst_hbm.at[w, j], idx_v)
        pltpu.async_copy(h_hbm.at[idx_v], rows_v, sem).wait()
        pltpu.sync_copy(rows_v, hd_hbm.at[pl.ds(off, CHUNK)])
        return ()

    lax.fori_loop(0, NCH, chunk, ())


@functools.partial(jax.jit, donate_argnums=())
def _sc_gather(h, src3, dst3):
    f = pl.kernel(
        _gather_body,
        out_type=(jax.ShapeDtypeStruct((E_PAD, D), jnp.float32),
                  jax.ShapeDtypeStruct((E_PAD, D), jnp.float32)),
        mesh=_sc_mesh(),
        scratch_types=[
            pltpu.VMEM((CHUNK,), jnp.int32),
            pltpu.VMEM((CHUNK, D), jnp.float32),
            pltpu.SemaphoreType.DMA,
        ],
        compiler_params=pltpu.CompilerParams(use_tc_tiling_on_sc=False),
    )
    return f(h, src3, dst3)


def _scatter_body(msg_hbm, dst_hbm, zero_hbm, agg_hbm,
                  idx_v, msg_v, acc_sh):
    c = lax.axis_index("c")
    s = lax.axis_index("s")
    w = c * NS + s
    base = w * EPW

    pltpu.sync_copy(zero_hbm.at[pl.ds(s * RPT, RPT)],
                    acc_sh.at[pl.ds(s * RPT, RPT)])
    plsc.subcore_barrier()

    def chunk(j, _):
        off = base + j * CHUNK
        pltpu.sync_copy(dst_hbm.at[w, j], idx_v)
        pltpu.sync_copy(msg_hbm.at[pl.ds(off, CHUNK)], msg_v)
        pltpu.sync_copy(msg_v, acc_sh.at[idx_v], add=True)
        return ()

    lax.fori_loop(0, NCH, chunk, ())
    plsc.subcore_barrier()
    pltpu.sync_copy(acc_sh.at[pl.ds(s * RPT, RPT)],
                    agg_hbm.at[c, pl.ds(s * RPT, RPT)])


@jax.jit
def _sc_scatter(msg, dst3, zeros):
    f = pl.kernel(
        _scatter_body,
        out_type=jax.ShapeDtypeStruct((NC, N_PAD, D), jnp.float32),
        mesh=_sc_mesh(),
        scratch_types=[
            pltpu.VMEM((CHUNK,), jnp.int32),
            pltpu.VMEM((CHUNK, D), jnp.float32),
            pltpu.VMEM_SHARED((N_PAD, D), jnp.float32),
        ],
        compiler_params=pltpu.CompilerParams(use_tc_tiling_on_sc=False),
    )
    return f(msg, dst3, zeros)


def _h0_body(nt_ref, emb_ref, out_ref):
    nt = nt_ref[:, 0]
    onehot = (nt[:, None] == lax.broadcasted_iota(jnp.int32, (1, N_ELEM), 1)
              ).astype(jnp.float32)
    out_ref[...] = jnp.dot(onehot, emb_ref[...],
                           preferred_element_type=jnp.float32)


def _tc_h0(nt2, embed):
    return pl.pallas_call(
        _h0_body,
        out_shape=jax.ShapeDtypeStruct((N, D), jnp.float32),
    )(nt2, embed)


def _edge_body(first, last, *refs):
    if first:
        (hs_ref, hd_ref, d_ref, wrbf_ref,
         w1a, w1b, w1c, wga, wgb, wgc, wea, web, wec) = refs[:13]
        d = d_ref[...]
        u = d * (1.0 / CUTOFF)
        n = (lax.broadcasted_iota(jnp.int32, (1, MAX_N), 1) + 1
             ).astype(jnp.float32)
        sbf = jnp.sin(n * (jnp.pi * u)) * ((2.0 / CUTOFF) ** 0.5 / d)
        env = jnp.where(u < 1.0, (1.0 - u ** 5) ** 2, 0.0)
        e = jnp.dot(sbf * env, wrbf_ref[...],
                    preferred_element_type=jnp.float32)
    else:
        (hs_ref, hd_ref, e_ref,
         w1a, w1b, w1c, wga, wgb, wgc, wea, web, wec) = refs[:12]
        e = e_ref[...]
    outs = refs[13:] if first else refs[12:]
    hs = hs_ref[...]
    hd = hd_ref[...]

    def three(wa, wb, wc):
        return (jnp.dot(hs, wa[...], preferred_element_type=jnp.float32)
                + jnp.dot(hd, wb[...], preferred_element_type=jnp.float32)
                + jnp.dot(e, wc[...], preferred_element_type=jnp.float32))

    z1 = three(w1a, w1b, w1c)
    zg = three(wga, wgb, wgc)
    outs[0][...] = z1 * jax.nn.sigmoid(z1) * jax.nn.sigmoid(zg)
    if not last:
        ze = three(wea, web, wec)
        outs[1][...] = e + ze * jax.nn.sigmoid(ze)


def _tc_edge(hs, hd, e_or_d, wrbf, wparts, first, last):
    grid = (E_PAD // EBLK,)
    row = pl.BlockSpec((EBLK, e_or_d.shape[1]), lambda i: (i, 0))
    full = lambda a: pl.BlockSpec(a.shape, lambda i: (0, 0))
    rowD = pl.BlockSpec((EBLK, D), lambda i: (i, 0))
    in_specs = [rowD, rowD, row]
    args = [hs, hd, e_or_d]
    if first:
        in_specs.append(full(wrbf))
        args.append(wrbf)
    for wp in wparts:
        in_specs.append(full(wp))
        args.append(wp)
    n_out = 1 if last else 2
    out_shape = [jax.ShapeDtypeStruct((E_PAD, D), jnp.float32)] * n_out
    out_specs = [rowD] * n_out
    res = pl.pallas_call(
        functools.partial(_edge_body, first, last),
        grid=grid,
        in_specs=in_specs,
        out_specs=out_specs,
        out_shape=out_shape,
        compiler_params=pltpu.CompilerParams(
            dimension_semantics=("arbitrary",)),
    )(*args)
    return res if not last else (res[0], None)


def _update_body(h_ref, agg_ref, out_ref):
    out_ref[...] = (h_ref[...] + agg_ref[0, 0:N, :] + agg_ref[1, 0:N, :])


def _tc_update(h, agg):
    return pl.pallas_call(
        _update_body,
        out_shape=jax.ShapeDtypeStruct((N, D), jnp.float32),
    )(h, agg)


def _readout_body(h_ref, agg_ref, wout_ref, out_ref):
    hn = h_ref[...] + agg_ref[0, 0:N, :] + agg_ref[1, 0:N, :]
    out_ref[...] = jnp.sum(hn * wout_ref[...], axis=1, keepdims=True)


def _tc_readout(h, agg, wout2):
    return pl.pallas_call(
        _readout_body,
        out_shape=jax.ShapeDtypeStruct((N, 1), jnp.float32),
    )(h, agg, wout2)


def kernel(node_types, edge_index, edge_dist, embed, W_rbf, W1, Wg, We, Wout):
    src = edge_index[0].astype(jnp.int32)
    dst = edge_index[1].astype(jnp.int32)
    pad = E_PAD - E
    src3 = jnp.concatenate([src, jnp.zeros((pad,), jnp.int32)]
                           ).reshape(NW, NCH, CHUNK)
    dst3 = jnp.concatenate([dst, jnp.full((pad,), N, jnp.int32)]
                           ).reshape(NW, NCH, CHUNK)
    d2 = jnp.concatenate([edge_dist, jnp.ones((pad,), jnp.float32)]
                         ).reshape(E_PAD, 1)
    zeros = jnp.zeros((N_PAD, D), jnp.float32)
    nt2 = node_types.astype(jnp.int32).reshape(N, 1)
    wout2 = Wout.reshape(1, D)

    h = _tc_h0(nt2, embed)
    e = d2
    agg = None
    for b in range(NBLOCKS):
        wparts = []
        for W in (W1, Wg, We):
            wb = W[b]
            wparts.extend([wb[0:D], wb[D:2 * D], wb[2 * D:3 * D]])
        hs, hd = _sc_gather(h, src3, dst3)
        first = b == 0
        last = b == NBLOCKS - 1
        msg, e = _tc_edge(hs, hd, e, W_rbf, wparts, first, last)
        agg = _sc_scatter(msg, dst3, zeros)
        if not last:
            h = _tc_update(h, agg)
    out = _tc_readout(h, agg, wout2)
    return out.reshape(N)


# speedup vs baseline: 1.6183x; 1.1428x over previous
import functools

import jax
import jax.numpy as jnp
from jax import lax
from jax.experimental import pallas as pl
from jax.experimental.pallas import tpu as pltpu
from jax.experimental.pallas import tpu_sc as plsc

N = 10000
E = 160000
D = 64
MAX_N = 9
NBLOCKS = 4
N_ELEM = 10
CUTOFF = 5.0

NC = 2
NS = 16
NW = NC * NS

CHUNK = 128
E_PAD = 163840
EPW = E_PAD // NW
NCH = EPW // CHUNK
N_PAD = 10240
RPT = N_PAD // NS

EBLK = 4096

def _sc_mesh():
    return plsc.VectorSubcoreMesh(core_axis_name="c", subcore_axis_name="s",
                                  num_cores=NC, num_subcores=NS)


G = 4
NGRP = NCH // G


def _gather_body(h_hbm, sd_hbm, hs_hbm, hd_hbm,
                 idx_v, buf_a, buf_b, sem_a, sem_b):
    c = lax.axis_index("c")
    s = lax.axis_index("s")
    w = c * NS + s
    base = w * EPW

    pltpu.sync_copy(sd_hbm.at[w], idx_v)

    bufs = ((buf_a, sem_a), (buf_b, sem_b))

    def fire(ep, g, buf, sem):
        return [pltpu.async_copy(h_hbm.at[idx_v.at[ep, g * G + k]],
                                 buf.at[pl.ds(k * CHUNK, CHUNK)], sem)
                for k in range(G)]

    pend = {0: fire(0, 0, *bufs[0])}
    for ep, out_hbm in enumerate((hs_hbm, hd_hbm)):
        for g in range(NGRP):
            pslot = g % 2
            if g + 1 < NGRP:
                pend[(g + 1) % 2] = fire(ep, g + 1, *bufs[(g + 1) % 2])
            elif ep == 0:
                pend[(g + 1) % 2] = fire(1, 0, *bufs[(g + 1) % 2])
            for hnd in pend.pop(pslot):
                hnd.wait()
            pltpu.sync_copy(bufs[pslot][0],
                            out_hbm.at[pl.ds(base + g * G * CHUNK, G * CHUNK)])


def _sc_gather(h, sd4):
    f = pl.kernel(
        _gather_body,
        out_type=(jax.ShapeDtypeStruct((E_PAD, D), jnp.float32),
                  jax.ShapeDtypeStruct((E_PAD, D), jnp.float32)),
        mesh=_sc_mesh(),
        scratch_types=[
            pltpu.VMEM((2, NCH, CHUNK), jnp.int32),
            pltpu.VMEM((G * CHUNK, D), jnp.float32),
            pltpu.VMEM((G * CHUNK, D), jnp.float32),
            pltpu.SemaphoreType.DMA,
            pltpu.SemaphoreType.DMA,
        ],
        compiler_params=pltpu.CompilerParams(use_tc_tiling_on_sc=False),
    )
    return f(h, sd4)


def _scatter_body(msg_hbm, dst_hbm, zero_hbm, agg_hbm,
                  idx_v, buf_a, buf_b, acc_sh, sem_a, sem_b):
    c = lax.axis_index("c")
    s = lax.axis_index("s")
    w = c * NS + s
    base = w * EPW

    pltpu.sync_copy(zero_hbm.at[pl.ds(s * RPT, RPT)],
                    acc_sh.at[pl.ds(s * RPT, RPT)])
    pltpu.sync_copy(dst_hbm.at[w], idx_v)
    plsc.subcore_barrier()

    bufs = ((buf_a, sem_a), (buf_b, sem_b))

    def fire(g, buf, sem):
        return pltpu.async_copy(
            msg_hbm.at[pl.ds(base + g * G * CHUNK, G * CHUNK)], buf, sem)

    pend = {0: fire(0, *bufs[0])}
    for g in range(NGRP):
        pslot = g % 2
        if g + 1 < NGRP:
            pend[(g + 1) % 2] = fire(g + 1, *bufs[(g + 1) % 2])
        pend.pop(pslot).wait()
        buf = bufs[pslot][0]
        for k in range(G):
            pltpu.sync_copy(buf.at[pl.ds(k * CHUNK, CHUNK)],
                            acc_sh.at[idx_v.at[g * G + k]], add=True)

    plsc.subcore_barrier()
    pltpu.sync_copy(acc_sh.at[pl.ds(s * RPT, RPT)],
                    agg_hbm.at[c, pl.ds(s * RPT, RPT)])


def _sc_scatter(msg, dst3, zeros):
    f = pl.kernel(
        _scatter_body,
        out_type=jax.ShapeDtypeStruct((NC, N_PAD, D), jnp.float32),
        mesh=_sc_mesh(),
        scratch_types=[
            pltpu.VMEM((NCH, CHUNK), jnp.int32),
            pltpu.VMEM((G * CHUNK, D), jnp.float32),
            pltpu.VMEM((G * CHUNK, D), jnp.float32),
            pltpu.VMEM_SHARED((N_PAD, D), jnp.float32),
            pltpu.SemaphoreType.DMA,
            pltpu.SemaphoreType.DMA,
        ],
        compiler_params=pltpu.CompilerParams(use_tc_tiling_on_sc=False),
    )
    return f(msg, dst3, zeros)


def _h0_body(nt_ref, emb_ref, out_ref):
    nt = nt_ref[:, 0]
    onehot = (nt[:, None] == lax.broadcasted_iota(jnp.int32, (1, N_ELEM), 1)
              ).astype(jnp.float32)
    out_ref[...] = jnp.dot(onehot, emb_ref[...],
                           preferred_element_type=jnp.float32)


def _tc_h0(nt2, embed):
    return pl.pallas_call(
        _h0_body,
        out_shape=jax.ShapeDtypeStruct((N, D), jnp.float32),
    )(nt2, embed)


def _edge_body(first, last, *refs):
    if first:
        (hs_ref, hd_ref, d_ref, wrbf_ref,
         w1a, w1b, w1c, wga, wgb, wgc, wea, web, wec) = refs[:13]
        d = d_ref[...]
        u = d * (1.0 / CUTOFF)
        n = (lax.broadcasted_iota(jnp.int32, (1, MAX_N), 1) + 1
             ).astype(jnp.float32)
        sbf = jnp.sin(n * (jnp.pi * u)) * ((2.0 / CUTOFF) ** 0.5 / d)
        env = jnp.where(u < 1.0, (1.0 - u ** 5) ** 2, 0.0)
        e = jnp.dot(sbf * env, wrbf_ref[...],
                    preferred_element_type=jnp.float32)
    else:
        (hs_ref, hd_ref, e_ref,
         w1a, w1b, w1c, wga, wgb, wgc, wea, web, wec) = refs[:12]
        e = e_ref[...]
    outs = refs[13:] if first else refs[12:]
    hs = hs_ref[...]
    hd = hd_ref[...]

    def three(wa, wb, wc):
        return (jnp.dot(hs, wa[...], preferred_element_type=jnp.float32)
                + jnp.dot(hd, wb[...], preferred_element_type=jnp.float32)
                + jnp.dot(e, wc[...], preferred_element_type=jnp.float32))

    z1 = three(w1a, w1b, w1c)
    zg = three(wga, wgb, wgc)
    outs[0][...] = z1 * jax.nn.sigmoid(z1) * jax.nn.sigmoid(zg)
    if not last:
        ze = three(wea, web, wec)
        outs[1][...] = e + ze * jax.nn.sigmoid(ze)


def _tc_edge(hs, hd, e_or_d, wrbf, wparts, first, last):
    grid = (E_PAD // EBLK,)
    row = pl.BlockSpec((EBLK, e_or_d.shape[1]), lambda i: (i, 0))
    full = lambda a: pl.BlockSpec(a.shape, lambda i: (0, 0))
    rowD = pl.BlockSpec((EBLK, D), lambda i: (i, 0))
    in_specs = [rowD, rowD, row]
    args = [hs, hd, e_or_d]
    if first:
        in_specs.append(full(wrbf))
        args.append(wrbf)
    for wp in wparts:
        in_specs.append(full(wp))
        args.append(wp)
    n_out = 1 if last else 2
    out_shape = [jax.ShapeDtypeStruct((E_PAD, D), jnp.float32)] * n_out
    out_specs = [rowD] * n_out
    res = pl.pallas_call(
        functools.partial(_edge_body, first, last),
        grid=grid,
        in_specs=in_specs,
        out_specs=out_specs,
        out_shape=out_shape,
        compiler_params=pltpu.CompilerParams(
            dimension_semantics=("arbitrary",)),
    )(*args)
    return res if not last else (res[0], None)


def _update_body(h_ref, agg_ref, out_ref):
    out_ref[...] = (h_ref[...] + agg_ref[0, 0:N, :] + agg_ref[1, 0:N, :])


def _tc_update(h, agg):
    return pl.pallas_call(
        _update_body,
        out_shape=jax.ShapeDtypeStruct((N, D), jnp.float32),
    )(h, agg)


def _readout_body(h_ref, agg_ref, wout_ref, out_ref):
    hn = h_ref[...] + agg_ref[0, 0:N, :] + agg_ref[1, 0:N, :]
    out_ref[...] = jnp.sum(hn * wout_ref[...], axis=1, keepdims=True)


def _tc_readout(h, agg, wout2):
    return pl.pallas_call(
        _readout_body,
        out_shape=jax.ShapeDtypeStruct((N, 1), jnp.float32),
    )(h, agg, wout2)


def kernel(node_types, edge_index, edge_dist, embed, W_rbf, W1, Wg, We, Wout):
    src = edge_index[0].astype(jnp.int32)
    dst = edge_index[1].astype(jnp.int32)
    pad = E_PAD - E
    src3 = jnp.concatenate([src, jnp.zeros((pad,), jnp.int32)]
                           ).reshape(NW, NCH, CHUNK)
    dst3 = jnp.concatenate([dst, jnp.full((pad,), N, jnp.int32)]
                           ).reshape(NW, NCH, CHUNK)
    sd4 = jnp.stack([src3, dst3], axis=1)
    d2 = jnp.concatenate([edge_dist, jnp.ones((pad,), jnp.float32)]
                         ).reshape(E_PAD, 1)
    zeros = jnp.zeros((N_PAD, D), jnp.float32)
    nt2 = node_types.astype(jnp.int32).reshape(N, 1)
    wout2 = Wout.reshape(1, D)

    h = _tc_h0(nt2, embed)
    e = d2
    agg = None
    for b in range(NBLOCKS):
        wparts = []
        for W in (W1, Wg, We):
            wb = W[b]
            wparts.extend([wb[0:D], wb[D:2 * D], wb[2 * D:3 * D]])
        hs, hd = _sc_gather(h, sd4)
        first = b == 0
        last = b == NBLOCKS - 1
        msg, e = _tc_edge(hs, hd, e, W_rbf, wparts, first, last)
        agg = _sc_scatter(msg, dst3, zeros)
        if not last:
            h = _tc_update(h, agg)
    out = _tc_readout(h, agg, wout2)
    return out.reshape(N)


# speedup vs baseline: 1.6917x; 1.0454x over previous
import functools

import jax
import jax.numpy as jnp
from jax import lax
from jax.experimental import pallas as pl
from jax.experimental.pallas import tpu as pltpu
from jax.experimental.pallas import tpu_sc as plsc

N = 10000
E = 160000
D = 64
MAX_N = 9
NBLOCKS = 4
N_ELEM = 10
CUTOFF = 5.0

NC = 2
NS = 16
NW = NC * NS

CHUNK = 128
E_PAD = 163840
EPW = E_PAD // NW
NCH = EPW // CHUNK
N_PAD = 10240
RPT = N_PAD // NS

EBLK = 4096

def _sc_mesh():
    return plsc.VectorSubcoreMesh(core_axis_name="c", subcore_axis_name="s",
                                  num_cores=NC, num_subcores=NS)


G = 4
NGRP = NCH // G


def _gather_body(h_hbm, sd_hbm, hs_hbm, hd_hbm,
                 idx_v, buf_a, buf_b, sem_a, sem_b):
    c = lax.axis_index("c")
    s = lax.axis_index("s")
    w = c * NS + s
    base = w * EPW

    pltpu.sync_copy(sd_hbm.at[w], idx_v)

    bufs = ((buf_a, sem_a), (buf_b, sem_b))

    def fire(ep, g, buf, sem):
        return [pltpu.async_copy(h_hbm.at[idx_v.at[ep, g * G + k]],
                                 buf.at[pl.ds(k * CHUNK, CHUNK)], sem)
                for k in range(G)]

    pend = {0: fire(0, 0, *bufs[0])}
    for ep, out_hbm in enumerate((hs_hbm, hd_hbm)):
        for g in range(NGRP):
            pslot = g % 2
            if g + 1 < NGRP:
                pend[(g + 1) % 2] = fire(ep, g + 1, *bufs[(g + 1) % 2])
            elif ep == 0:
                pend[(g + 1) % 2] = fire(1, 0, *bufs[(g + 1) % 2])
            for hnd in pend.pop(pslot):
                hnd.wait()
            pltpu.sync_copy(bufs[pslot][0],
                            out_hbm.at[pl.ds(base + g * G * CHUNK, G * CHUNK)])


def _sc_gather(h, sd4):
    f = pl.kernel(
        _gather_body,
        out_type=(jax.ShapeDtypeStruct((E_PAD, D), jnp.bfloat16),
                  jax.ShapeDtypeStruct((E_PAD, D), jnp.bfloat16)),
        mesh=_sc_mesh(),
        scratch_types=[
            pltpu.VMEM((2, NCH, CHUNK), jnp.int32),
            pltpu.VMEM((G * CHUNK, D), jnp.bfloat16),
            pltpu.VMEM((G * CHUNK, D), jnp.bfloat16),
            pltpu.SemaphoreType.DMA,
            pltpu.SemaphoreType.DMA,
        ],
        compiler_params=pltpu.CompilerParams(use_tc_tiling_on_sc=False),
    )
    return f(h, sd4)


def _scatter_body(msg_hbm, dst_hbm, zero_hbm, agg_hbm,
                  idx_v, buf_a, buf_b, acc_sh, sem_a, sem_b):
    c = lax.axis_index("c")
    s = lax.axis_index("s")
    w = c * NS + s
    base = w * EPW

    pltpu.sync_copy(zero_hbm.at[pl.ds(s * RPT, RPT)],
                    acc_sh.at[pl.ds(s * RPT, RPT)])
    pltpu.sync_copy(dst_hbm.at[w], idx_v)
    plsc.subcore_barrier()

    bufs = ((buf_a, sem_a), (buf_b, sem_b))

    def fire(g, buf, sem):
        return pltpu.async_copy(
            msg_hbm.at[pl.ds(base + g * G * CHUNK, G * CHUNK)], buf, sem)

    pend = {0: fire(0, *bufs[0])}
    for g in range(NGRP):
        pslot = g % 2
        if g + 1 < NGRP:
            pend[(g + 1) % 2] = fire(g + 1, *bufs[(g + 1) % 2])
        pend.pop(pslot).wait()
        buf = bufs[pslot][0]
        for k in range(G):
            pltpu.sync_copy(buf.at[pl.ds(k * CHUNK, CHUNK)],
                            acc_sh.at[idx_v.at[g * G + k]], add=True)

    plsc.subcore_barrier()
    pltpu.sync_copy(acc_sh.at[pl.ds(s * RPT, RPT)],
                    agg_hbm.at[c, pl.ds(s * RPT, RPT)])


def _sc_scatter(msg, dst3, zeros):
    f = pl.kernel(
        _scatter_body,
        out_type=jax.ShapeDtypeStruct((NC, N_PAD, D), jnp.float32),
        mesh=_sc_mesh(),
        scratch_types=[
            pltpu.VMEM((NCH, CHUNK), jnp.int32),
            pltpu.VMEM((G * CHUNK, D), jnp.float32),
            pltpu.VMEM((G * CHUNK, D), jnp.float32),
            pltpu.VMEM_SHARED((N_PAD, D), jnp.float32),
            pltpu.SemaphoreType.DMA,
            pltpu.SemaphoreType.DMA,
        ],
        compiler_params=pltpu.CompilerParams(use_tc_tiling_on_sc=False),
    )
    return f(msg, dst3, zeros)


def _h0_body(nt_ref, emb_ref, out_ref, out16_ref):
    nt = nt_ref[:, 0]
    onehot = (nt[:, None] == lax.broadcasted_iota(jnp.int32, (1, N_ELEM), 1)
              ).astype(jnp.float32)
    h = jnp.dot(onehot, emb_ref[...], preferred_element_type=jnp.float32)
    out_ref[...] = h
    out16_ref[...] = h.astype(jnp.bfloat16)


def _tc_h0(nt2, embed):
    return pl.pallas_call(
        _h0_body,
        out_shape=(jax.ShapeDtypeStruct((N, D), jnp.float32),
                   jax.ShapeDtypeStruct((N, D), jnp.bfloat16)),
    )(nt2, embed)


def _edge_body(first, last, *refs):
    if first:
        (hs_ref, hd_ref, d_ref, wrbf_ref,
         w1a, w1b, w1c, wga, wgb, wgc, wea, web, wec) = refs[:13]
        d = d_ref[...]
        u = d * (1.0 / CUTOFF)
        n = (lax.broadcasted_iota(jnp.int32, (1, MAX_N), 1) + 1
             ).astype(jnp.float32)
        sbf = jnp.sin(n * (jnp.pi * u)) * ((2.0 / CUTOFF) ** 0.5 / d)
        env = jnp.where(u < 1.0, (1.0 - u ** 5) ** 2, 0.0)
        e = jnp.dot(sbf * env, wrbf_ref[...],
                    preferred_element_type=jnp.float32)
    else:
        (hs_ref, hd_ref, e_ref,
         w1a, w1b, w1c, wga, wgb, wgc, wea, web, wec) = refs[:12]
        e = e_ref[...]
    outs = refs[13:] if first else refs[12:]
    hs = hs_ref[...]
    hd = hd_ref[...]
    e16 = e.astype(jnp.bfloat16)

    def three(wa, wb, wc):
        return (jnp.dot(hs, wa[...].astype(jnp.bfloat16),
                        preferred_element_type=jnp.float32)
                + jnp.dot(hd, wb[...].astype(jnp.bfloat16),
                          preferred_element_type=jnp.float32)
                + jnp.dot(e16, wc[...].astype(jnp.bfloat16),
                          preferred_element_type=jnp.float32))

    z1 = three(w1a, w1b, w1c)
    zg = three(wga, wgb, wgc)
    outs[0][...] = z1 * jax.nn.sigmoid(z1) * jax.nn.sigmoid(zg)
    if not last:
        ze = three(wea, web, wec)
        outs[1][...] = e + ze * jax.nn.sigmoid(ze)


def _tc_edge(hs, hd, e_or_d, wrbf, wparts, first, last):
    grid = (E_PAD // EBLK,)
    row = pl.BlockSpec((EBLK, e_or_d.shape[1]), lambda i: (i, 0))
    full = lambda a: pl.BlockSpec(a.shape, lambda i: (0, 0))
    rowD = pl.BlockSpec((EBLK, D), lambda i: (i, 0))
    in_specs = [rowD, rowD, row]
    args = [hs, hd, e_or_d]
    if first:
        in_specs.append(full(wrbf))
        args.append(wrbf)
    for wp in wparts:
        in_specs.append(full(wp))
        args.append(wp)
    n_out = 1 if last else 2
    out_shape = [jax.ShapeDtypeStruct((E_PAD, D), jnp.float32)] * n_out
    out_specs = [rowD] * n_out
    res = pl.pallas_call(
        functools.partial(_edge_body, first, last),
        grid=grid,
        in_specs=in_specs,
        out_specs=out_specs,
        out_shape=out_shape,
        compiler_params=pltpu.CompilerParams(
            dimension_semantics=("arbitrary",)),
    )(*args)
    return res if not last else (res[0], None)


def _update_body(h_ref, agg_ref, out_ref, out16_ref):
    hn = h_ref[...] + agg_ref[0, 0:N, :] + agg_ref[1, 0:N, :]
    out_ref[...] = hn
    out16_ref[...] = hn.astype(jnp.bfloat16)


def _tc_update(h, agg):
    return pl.pallas_call(
        _update_body,
        out_shape=(jax.ShapeDtypeStruct((N, D), jnp.float32),
                   jax.ShapeDtypeStruct((N, D), jnp.bfloat16)),
    )(h, agg)


def _readout_body(h_ref, agg_ref, wout_ref, out_ref):
    hn = h_ref[...] + agg_ref[0, 0:N, :] + agg_ref[1, 0:N, :]
    out_ref[...] = jnp.sum(hn * wout_ref[...], axis=1, keepdims=True)


def _tc_readout(h, agg, wout2):
    return pl.pallas_call(
        _readout_body,
        out_shape=jax.ShapeDtypeStruct((N, 1), jnp.float32),
    )(h, agg, wout2)


def kernel(node_types, edge_index, edge_dist, embed, W_rbf, W1, Wg, We, Wout):
    src = edge_index[0].astype(jnp.int32)
    dst = edge_index[1].astype(jnp.int32)
    pad = E_PAD - E
    src3 = jnp.concatenate([src, jnp.zeros((pad,), jnp.int32)]
                           ).reshape(NW, NCH, CHUNK)
    dst3 = jnp.concatenate([dst, jnp.full((pad,), N, jnp.int32)]
                           ).reshape(NW, NCH, CHUNK)
    sd4 = jnp.stack([src3, dst3], axis=1)
    d2 = jnp.concatenate([edge_dist, jnp.ones((pad,), jnp.float32)]
                         ).reshape(E_PAD, 1)
    zeros = jnp.zeros((N_PAD, D), jnp.float32)
    nt2 = node_types.astype(jnp.int32).reshape(N, 1)
    wout2 = Wout.reshape(1, D)

    h, h16 = _tc_h0(nt2, embed)
    e = d2
    agg = None
    for b in range(NBLOCKS):
        wparts = []
        for W in (W1, Wg, We):
            wb = W[b]
            wparts.extend([wb[0:D], wb[D:2 * D], wb[2 * D:3 * D]])
        hs, hd = _sc_gather(h16, sd4)
        first = b == 0
        last = b == NBLOCKS - 1
        msg, e = _tc_edge(hs, hd, e, W_rbf, wparts, first, last)
        agg = _sc_scatter(msg, dst3, zeros)
        if not last:
            h, h16 = _tc_update(h, agg)
    out = _tc_readout(h, agg, wout2)
    return out.reshape(N)
